# baseline (speedup 1.0000x reference)
import functools

import jax
import jax.numpy as jnp
from jax import lax
from jax.experimental import pallas as pl
from jax.experimental.pallas import tpu as pltpu
from jax.experimental.pallas import tpu_sc as plsc

_G = 2
_V = 320
_D = 128
_H = 512
_B = 8
_S = 1024
_TOK = _B * _S
_TBLK = 1024
_NBLK = _TOK // _TBLK


def _tc_body(hs_ref, w0_ref, w1_ref, b_ref, i0_ref, i1_ref, perp_ref,
             counts_ref):
    i = pl.program_id(0)

    @pl.when(i == 0)
    def _init():
        counts_ref[...] = jnp.zeros_like(counts_ref)

    iota_v = lax.broadcasted_iota(jnp.int32, (_TBLK, _V), 1)
    ones_row = jnp.ones((1, _TBLK), jnp.float32)
    for g, (w_ref, out_ref) in enumerate(((w0_ref, i0_ref), (w1_ref, i1_ref))):
        lg = (
            jnp.dot(hs_ref[...], w_ref[...], preferred_element_type=jnp.float32)
            + b_ref[g : g + 1, :]
        )
        m = jnp.max(lg, axis=1, keepdims=True)
        idx = jnp.min(jnp.where(lg == m, iota_v, _V), axis=1).astype(jnp.int32)
        out_ref[...] = idx + g * _V
        onehot = (iota_v == idx[:, None]).astype(jnp.float32)
        counts_ref[g : g + 1, :] += jnp.dot(
            ones_row, onehot, preferred_element_type=jnp.float32
        )

    @pl.when(i == _NBLK - 1)
    def _fin():
        p = counts_ref[...] * (1.0 / _TOK)
        ent = -jnp.sum(p * jnp.log(p + 1e-7), axis=1)
        perp_ref[...] = jnp.broadcast_to(jnp.sum(jnp.exp(ent)), (1, 1))


def _tc_call(hs, W0, W1, b2d):
    return pl.pallas_call(
        _tc_body,
        grid=(_NBLK,),
        in_specs=[
            pl.BlockSpec((_TBLK, _H), lambda i: (i, 0)),
            pl.BlockSpec((_H, _V), lambda i: (0, 0)),
            pl.BlockSpec((_H, _V), lambda i: (0, 0)),
            pl.BlockSpec((_G, _V), lambda i: (0, 0)),
        ],
        out_specs=[
            pl.BlockSpec((_TBLK,), lambda i: (i,)),
            pl.BlockSpec((_TBLK,), lambda i: (i,)),
            pl.BlockSpec((1, 1), lambda i: (0, 0)),
        ],
        out_shape=[
            jax.ShapeDtypeStruct((_TOK,), jnp.int32),
            jax.ShapeDtypeStruct((_TOK,), jnp.int32),
            jax.ShapeDtypeStruct((1, 1), jnp.float32),
        ],
        scratch_shapes=[pltpu.VMEM((_G, _V), jnp.float32)],
    )(hs, W0, W1, b2d)


_NW = 32
_TOK_PER_W = _TOK // _NW


@functools.lru_cache(maxsize=1)
def _make_sc_gather():
    @functools.partial(
        pl.kernel,
        mesh=plsc.VectorSubcoreMesh(core_axis_name="c", subcore_axis_name="s"),
        out_type=jax.ShapeDtypeStruct((_B, _S, _G * _D), jnp.float32),
        scratch_types=[
            pltpu.VMEM((_TOK_PER_W,), jnp.int32),
            pltpu.VMEM((_TOK_PER_W,), jnp.int32),
            pltpu.VMEM((_TOK_PER_W, _D), jnp.float32),
            pltpu.VMEM((_TOK_PER_W, _D), jnp.float32),
            pltpu.VMEM_SHARED((_G * _V, _D), jnp.float32),
            pltpu.SemaphoreType.DMA,
            pltpu.SemaphoreType.DMA,
        ],
    )
    def _sc_gather(table_hbm, i0_hbm, i1_hbm, out_hbm, i0_v, i1_v, r0_v, r1_v,
                   table_sh, sem, sem1):
        sid = lax.axis_index("s")
        wid = sid * 2 + lax.axis_index("c")
        base = wid * _TOK_PER_W
        b = base // _S
        s0 = base % _S

        @pl.when(sid == 0)
        def _stage():
            pltpu.sync_copy(table_hbm, table_sh)

        pltpu.sync_copy(i0_hbm.at[pl.ds(base, _TOK_PER_W)], i0_v)
        pltpu.sync_copy(i1_hbm.at[pl.ds(base, _TOK_PER_W)], i1_v)
        plsc.subcore_barrier()
        c0 = pltpu.async_copy(table_sh.at[i0_v], r0_v, sem)
        c1 = pltpu.async_copy(table_sh.at[i1_v], r1_v, sem1)
        c0.wait()
        pltpu.sync_copy(r0_v, out_hbm.at[b, pl.ds(s0, _TOK_PER_W), pl.ds(0, _D)])
        c1.wait()
        pltpu.sync_copy(r1_v, out_hbm.at[b, pl.ds(s0, _TOK_PER_W), pl.ds(_D, _D)])

    return _sc_gather


def kernel(hidden_states, W, b, codevectors):
    bsz, seq, _ = hidden_states.shape
    hs = hidden_states.reshape(bsz * seq, _H)
    W0 = W[:, :_V]
    W1 = W[:, _V:]
    i0, i1, perp = _tc_call(hs, W0, W1, b.reshape(_G, _V))
    table = codevectors.reshape(_G * _V, _D)
    out = _make_sc_gather()(table, i0, i1)
    return out, perp[0, 0]


# speedup vs baseline: 1.0007x; 1.0007x over previous
import functools

import jax
import jax.numpy as jnp
from jax import lax
from jax.experimental import pallas as pl
from jax.experimental.pallas import tpu as pltpu
from jax.experimental.pallas import tpu_sc as plsc

_G = 2
_V = 320
_D = 128
_H = 512
_B = 8
_S = 1024
_TOK = _B * _S
_TBLK = 1024
_NBLK = _TOK // _TBLK


def _tc_body(hs_ref, w0_ref, w1_ref, b_ref, i0_ref, i1_ref, perp_ref,
             counts_ref):
    i = pl.program_id(0)

    @pl.when(i == 0)
    def _init():
        counts_ref[...] = jnp.zeros_like(counts_ref)

    iota_v = lax.broadcasted_iota(jnp.int32, (_TBLK, _V), 1)
    ones_row = jnp.ones((1, _TBLK), jnp.float32)
    for g, (w_ref, out_ref) in enumerate(((w0_ref, i0_ref), (w1_ref, i1_ref))):
        lg = (
            jnp.dot(hs_ref[...], w_ref[...], preferred_element_type=jnp.float32)
            + b_ref[g : g + 1, :]
        )
        m = jnp.max(lg, axis=1, keepdims=True)
        idx = jnp.min(jnp.where(lg == m, iota_v, _V), axis=1).astype(jnp.int32)
        out_ref[...] = idx + g * _V
        onehot = (iota_v == idx[:, None]).astype(jnp.float32)
        counts_ref[g : g + 1, :] += jnp.dot(
            ones_row, onehot, preferred_element_type=jnp.float32
        )

    @pl.when(i == _NBLK - 1)
    def _fin():
        p = counts_ref[...] * (1.0 / _TOK)
        ent = -jnp.sum(p * jnp.log(p + 1e-7), axis=1)
        perp_ref[...] = jnp.broadcast_to(jnp.sum(jnp.exp(ent)), (1, 1))


def _tc_call(hs, W0, W1, b2d):
    return pl.pallas_call(
        _tc_body,
        grid=(_NBLK,),
        in_specs=[
            pl.BlockSpec((_TBLK, _H), lambda i: (i, 0)),
            pl.BlockSpec((_H, _V), lambda i: (0, 0)),
            pl.BlockSpec((_H, _V), lambda i: (0, 0)),
            pl.BlockSpec((_G, _V), lambda i: (0, 0)),
        ],
        out_specs=[
            pl.BlockSpec((_TBLK,), lambda i: (i,)),
            pl.BlockSpec((_TBLK,), lambda i: (i,)),
            pl.BlockSpec((1, 1), lambda i: (0, 0)),
        ],
        out_shape=[
            jax.ShapeDtypeStruct((_TOK,), jnp.int32),
            jax.ShapeDtypeStruct((_TOK,), jnp.int32),
            jax.ShapeDtypeStruct((1, 1), jnp.float32),
        ],
        scratch_shapes=[pltpu.VMEM((_G, _V), jnp.float32)],
    )(hs, W0, W1, b2d)


_NW = 32
_TOK_PER_W = _TOK // _NW


@functools.lru_cache(maxsize=1)
def _make_sc_gather():
    @functools.partial(
        pl.kernel,
        mesh=plsc.VectorSubcoreMesh(core_axis_name="c", subcore_axis_name="s"),
        out_type=jax.ShapeDtypeStruct((_B, _S, _G * _D), jnp.float32),
        scratch_types=[
            pltpu.VMEM((_TOK_PER_W,), jnp.int32),
            pltpu.VMEM((_TOK_PER_W,), jnp.int32),
            pltpu.VMEM((_TOK_PER_W, _D), jnp.float32),
            pltpu.VMEM((_TOK_PER_W, _D), jnp.float32),
            pltpu.VMEM_SHARED((_G * _V, _D), jnp.float32),
            pltpu.SemaphoreType.DMA,
        ],
    )
    def _sc_gather(table_hbm, i0_hbm, i1_hbm, out_hbm, i0_v, i1_v, r0_v, r1_v,
                   table_sh, sem):
        sid = lax.axis_index("s")
        wid = sid * 2 + lax.axis_index("c")
        base = wid * _TOK_PER_W
        b = base // _S
        s0 = base % _S

        @pl.when(sid == 0)
        def _stage():
            pltpu.sync_copy(table_hbm, table_sh)

        pltpu.sync_copy(i0_hbm.at[pl.ds(base, _TOK_PER_W)], i0_v)
        pltpu.sync_copy(i1_hbm.at[pl.ds(base, _TOK_PER_W)], i1_v)
        plsc.subcore_barrier()
        c0 = pltpu.async_copy(table_sh.at[i0_v], r0_v, sem)
        c1 = pltpu.async_copy(table_sh.at[i1_v], r1_v, sem)
        c0.wait()
        c1.wait()
        pltpu.sync_copy(r0_v, out_hbm.at[b, pl.ds(s0, _TOK_PER_W), pl.ds(0, _D)])
        pltpu.sync_copy(r1_v, out_hbm.at[b, pl.ds(s0, _TOK_PER_W), pl.ds(_D, _D)])

    return _sc_gather


def kernel(hidden_states, W, b, codevectors):
    bsz, seq, _ = hidden_states.shape
    hs = hidden_states.reshape(bsz * seq, _H)
    W0 = W[:, :_V]
    W1 = W[:, _V:]
    i0, i1, perp = _tc_call(hs, W0, W1, b.reshape(_G, _V))
    table = codevectors.reshape(_G * _V, _D)
    out = _make_sc_gather()(table, i0, i1)
    return out, perp[0, 0]
